# baseline (speedup 1.0000x reference)
import functools

import jax
import jax.numpy as jnp
from jax import lax
from jax.experimental import pallas as pl
from jax.experimental.pallas import tpu as pltpu
from jax.experimental.pallas import tpu_sc as plsc

D = 128
B = 512
WL = 40
WIN = 5
NEG = 5
T = B * WL
NNEG = T * NEG
NPOS = B * 2 * sum(WL - d for d in range(1, WIN + 1))

NC = 2
NS = 16
NW = NC * NS
TPW = T // NW
NEGPW = NNEG // NW
CH = 128
NCH_TOK = TPW // CH

NTOK_CH = 16
NEG_CH = NTOK_CH * NEG
NCH_NEG = TPW // NTOK_CH
PPR = NEG_CH // 8
PPG = 4
PP_ROWS = NNEG // 8

_SC_MESH = dict(core_axis_name="c", subcore_axis_name="s",
                num_cores=NC, num_subcores=NS)


def _worker_id():
    return lax.axis_index("s") * NC + lax.axis_index("c")


def _sc_tok_body(walk_hbm, node_hbm, ctx_hbm, nodeg_hbm, ctxg_hbm,
                 walk_v, rows_v, gs0, gs1, gs2, gs3, os0, os1, os2, os3):
    gsems = (gs0, gs1, gs2, gs3)
    osems = (os0, os1, os2, os3)
    wid = _worker_id()
    tbase = wid * TPW

    pltpu.sync_copy(walk_hbm.at[pl.ds(tbase, TPW)], walk_v)

    def gather(j, b):
        idx = walk_v.at[pl.ds((j // 2) * CH, CH)]
        tab = node_hbm if j % 2 == 0 else ctx_hbm
        pltpu.async_copy(tab.at[idx], rows_v.at[b], gsems[b])

    def out(j, b):
        dst = nodeg_hbm if j % 2 == 0 else ctxg_hbm
        pltpu.async_copy(
            rows_v.at[b], dst.at[pl.ds(tbase + (j // 2) * CH, CH)], osems[b])

    for j in range(4):
        gather(j, j)
    for j in range(2 * NCH_TOK):
        b = j % 4
        pltpu.make_async_copy(
            node_hbm.at[pl.ds(0, CH)], rows_v.at[b], gsems[b]).wait()
        out(j, b)
        pltpu.make_async_copy(
            rows_v.at[b], nodeg_hbm.at[pl.ds(0, CH)], osems[b]).wait()
        if j + 4 < 2 * NCH_TOK:
            gather(j + 4, b)


@functools.cache
def _sc_tok():
    return pl.kernel(
        _sc_tok_body,
        out_type=(
            jax.ShapeDtypeStruct((T, D), jnp.float32),
            jax.ShapeDtypeStruct((T, D), jnp.float32),
        ),
        mesh=plsc.VectorSubcoreMesh(**_SC_MESH),
        scratch_types=(
            pltpu.VMEM((TPW,), jnp.int32),
            pltpu.VMEM((4, CH, D), jnp.float32),
        ) + (pltpu.SemaphoreType.DMA,) * 8,
    )


TOK_HALF = TPW // 2
CH_HALF = NCH_NEG // 2
NRING = 4


def _sc_neg_body(walk_hbm, negidx_hbm, ctx_hbm, nodeg_hbm, pp_hbm,
                 negidx_v, nid_v, node_v, nrow_v, pp_v,
                 nsem, csem, psem, ng0, ng1, ng2, ng3):
    ngsems = (ng0, ng1, ng2, ng3)
    wid = _worker_id()
    tbase = wid * TPW
    nbase = wid * NEGPW
    pbase = wid * (NEGPW // 8)

    pltpu.async_copy(nodeg_hbm.at[pl.ds(tbase, TOK_HALF)], node_v, nsem)

    pltpu.sync_copy(negidx_hbm.at[pl.ds(nbase, NEGPW)], negidx_v)

    @pl.loop(0, NEGPW // CH)
    def _compose(c):
        pltpu.async_copy(walk_hbm.at[negidx_v.at[pl.ds(c * CH, CH)]],
                         nid_v.at[pl.ds(c * CH, CH)], csem)

    pltpu.make_async_copy(
        walk_hbm.at[pl.ds(0, NEGPW)], nid_v, csem).wait()

    def neg_gather(c, b):
        pltpu.async_copy(
            ctx_hbm.at[nid_v.at[pl.ds(c * NEG_CH, NEG_CH)]],
            nrow_v.at[b], ngsems[b])

    for c in range(NRING):
        neg_gather(c, c)

    pltpu.make_async_copy(
        nodeg_hbm.at[pl.ds(0, TOK_HALF)], node_v, nsem).wait()

    for h in range(2):
        @pl.loop(0, CH_HALF // PPG)
        def _neg_group(g):
            for cc in range(PPG):
                cl = g * PPG + cc
                c = h * CH_HALF + cl
                rb = cc
                pltpu.make_async_copy(
                    ctx_hbm.at[pl.ds(0, NEG_CH)],
                    nrow_v.at[rb], ngsems[rb]).wait()

                @pl.loop(0, NTOK_CH)
                def _tok(tt):
                    trow = cl * NTOK_CH + tt
                    nd = [node_v[trow, pl.ds(q * 16, 16)] for q in range(8)]
                    for k in range(NEG):
                        r = tt * NEG + k
                        acc0 = nd[0] * nrow_v[rb, r, pl.ds(0, 16)]
                        acc1 = nd[1] * nrow_v[rb, r, pl.ds(16, 16)]
                        for q in range(2, 8, 2):
                            acc0 += nd[q] * nrow_v[rb, r, pl.ds(q * 16, 16)]
                            acc1 += (nd[q + 1]
                                     * nrow_v[rb, r, pl.ds(q * 16 + 16, 16)])
                        pp_v[cc * PPR + lax.shift_right_logical(r, 3),
                             pl.ds(lax.shift_left(lax.bitwise_and(r, 7), 4),
                                   16)] = acc0 + acc1

                nc = c + NRING

                @pl.when(nc < NCH_NEG)
                def _():
                    neg_gather(nc, rb)

            pltpu.async_copy(
                pp_v,
                pp_hbm.at[pl.ds(pbase + h * (CH_HALF * PPR) + g * (PPG * PPR),
                                PPG * PPR)], psem)
            pltpu.make_async_copy(
                pp_v, pp_hbm.at[pl.ds(0, PPG * PPR)], psem).wait()

        if h == 0:
            pltpu.sync_copy(
                nodeg_hbm.at[pl.ds(tbase + TOK_HALF, TOK_HALF)], node_v)


@functools.cache
def _sc_neg():
    return pl.kernel(
        _sc_neg_body,
        out_type=jax.ShapeDtypeStruct((PP_ROWS, D), jnp.float32),
        mesh=plsc.VectorSubcoreMesh(**_SC_MESH),
        scratch_types=(
            pltpu.VMEM((NEGPW,), jnp.int32),
            pltpu.VMEM((NEGPW,), jnp.int32),
            pltpu.VMEM((TOK_HALF, D), jnp.float32),
            pltpu.VMEM((NRING, NEG_CH, D), jnp.float32),
            pltpu.VMEM((PPG * PPR, D), jnp.float32),
        ) + (pltpu.SemaphoreType.DMA,) * 7,
    )


TC_GRID = 32
TB = T // TC_GRID
WPB = TB // WL
NEG_GRID = 8
PPB = PP_ROWS // NEG_GRID


def _nls(score):
    return jnp.log1p(jnp.exp(-jnp.clip(score, -6.0, 6.0)))


def _tc_pos_body(node_ref, ctx_ref, acc_ref):
    i = pl.program_id(0)
    nb = node_ref[...].astype(jnp.bfloat16)
    cb = ctx_ref[...].astype(jnp.bfloat16)
    jj = lax.broadcasted_iota(jnp.int32, (WL, WL), 0)
    ii = lax.broadcasted_iota(jnp.int32, (WL, WL), 1)
    dd = ii - jj
    band = (dd != 0) & (dd >= -WIN) & (dd <= WIN)
    pos_sum = jnp.float32(0.0)
    for w in range(WPB):
        a = nb[w * WL:(w + 1) * WL]
        b = cb[w * WL:(w + 1) * WL]
        g = lax.dot_general(a, b, (((1,), (1,)), ((), ())),
                            preferred_element_type=jnp.float32)
        pos_sum += jnp.sum(jnp.where(band, _nls(g), 0.0))

    @pl.when(i == 0)
    def _init():
        acc_ref[0, 0] = jnp.float32(0.0)

    acc_ref[0, 0] += pos_sum


_tc_pos = pl.pallas_call(
    _tc_pos_body,
    grid=(TC_GRID,),
    in_specs=[
        pl.BlockSpec((TB, D), lambda i: (i, 0)),
        pl.BlockSpec((TB, D), lambda i: (i, 0)),
    ],
    out_specs=pl.BlockSpec(memory_space=pltpu.SMEM),
    out_shape=jax.ShapeDtypeStruct((1, 1), jnp.float32),
)


def _tc_neg_body(pp_ref, acc_ref):
    i = pl.program_id(0)
    seg = (lax.broadcasted_iota(jnp.int32, (D, 8), 0) // 16
           == lax.broadcasted_iota(jnp.int32, (D, 8), 1)).astype(jnp.float32)
    s8 = lax.dot_general(pp_ref[...], seg, (((1,), (0,)), ((), ())),
                         preferred_element_type=jnp.float32)
    neg_sum = jnp.sum(_nls(-s8))

    @pl.when(i == 0)
    def _init():
        acc_ref[0, 0] = jnp.float32(0.0)

    acc_ref[0, 0] += neg_sum


_tc_neg = pl.pallas_call(
    _tc_neg_body,
    grid=(NEG_GRID,),
    in_specs=[pl.BlockSpec((PPB, D), lambda i: (i, 0))],
    out_specs=pl.BlockSpec(memory_space=pltpu.SMEM),
    out_shape=jax.ShapeDtypeStruct((1, 1), jnp.float32),
)


def kernel(batch_walk, neg_idx_list_dst, node_embed_weight, context_embed_weight):
    flat_walk = batch_walk.reshape(-1)
    node_g, ctx_g = _sc_tok()(flat_walk, node_embed_weight, context_embed_weight)
    pp = _sc_neg()(flat_walk, neg_idx_list_dst, context_embed_weight, node_g)
    pos_acc = _tc_pos(node_g, ctx_g)
    neg_acc = _tc_neg(pp)
    pos_loss = pos_acc[0, 0] / NPOS
    neg_loss = neg_acc[0, 0] * (NEG * 1.0) / NNEG
    return pos_loss + neg_loss


# speedup vs baseline: 1.0158x; 1.0158x over previous
import functools

import jax
import jax.numpy as jnp
from jax import lax
from jax.experimental import pallas as pl
from jax.experimental.pallas import tpu as pltpu
from jax.experimental.pallas import tpu_sc as plsc

D = 128
B = 512
WL = 40
WIN = 5
NEG = 5
T = B * WL
NNEG = T * NEG
NPOS = B * 2 * sum(WL - d for d in range(1, WIN + 1))

NC = 2
NS = 16
NW = NC * NS
TPW = T // NW
NEGPW = NNEG // NW
CH = 128
NCH_TOK = TPW // CH

NTOK_CH = 16
NEG_CH = NTOK_CH * NEG
NCH_NEG = TPW // NTOK_CH
PPR = NEG_CH // 8
PPG = 4
PP_ROWS = NNEG // 8

_SC_MESH = dict(core_axis_name="c", subcore_axis_name="s",
                num_cores=NC, num_subcores=NS)


def _worker_id():
    return lax.axis_index("s") * NC + lax.axis_index("c")


def _sc_tok_body(walk_hbm, node_hbm, ctx_hbm, nodeg_hbm, ctxg_hbm,
                 walk_v, rows_v, gs0, gs1, gs2, gs3, os0, os1, os2, os3):
    gsems = (gs0, gs1, gs2, gs3)
    osems = (os0, os1, os2, os3)
    wid = _worker_id()
    tbase = wid * TPW

    pltpu.sync_copy(walk_hbm.at[pl.ds(tbase, TPW)], walk_v)

    def gather(j, b):
        idx = walk_v.at[pl.ds((j // 2) * CH, CH)]
        tab = node_hbm if j % 2 == 0 else ctx_hbm
        pltpu.async_copy(tab.at[idx], rows_v.at[b], gsems[b])

    def out(j, b):
        dst = nodeg_hbm if j % 2 == 0 else ctxg_hbm
        pltpu.async_copy(
            rows_v.at[b], dst.at[pl.ds(tbase + (j // 2) * CH, CH)], osems[b])

    for j in range(4):
        gather(j, j)
    for j in range(2 * NCH_TOK):
        b = j % 4
        pltpu.make_async_copy(
            node_hbm.at[pl.ds(0, CH)], rows_v.at[b], gsems[b]).wait()
        out(j, b)
        pltpu.make_async_copy(
            rows_v.at[b], nodeg_hbm.at[pl.ds(0, CH)], osems[b]).wait()
        if j + 4 < 2 * NCH_TOK:
            gather(j + 4, b)


@functools.cache
def _sc_tok():
    return pl.kernel(
        _sc_tok_body,
        out_type=(
            jax.ShapeDtypeStruct((T, D), jnp.float32),
            jax.ShapeDtypeStruct((T, D), jnp.float32),
        ),
        mesh=plsc.VectorSubcoreMesh(**_SC_MESH),
        scratch_types=(
            pltpu.VMEM((TPW,), jnp.int32),
            pltpu.VMEM((4, CH, D), jnp.float32),
        ) + (pltpu.SemaphoreType.DMA,) * 8,
    )


def _sc_neg_body(walk_hbm, negidx_hbm, ctx_hbm, nodeg_hbm, pp_hbm,
                 negidx_v, nid_v, node_v, nrow_v, pp_v,
                 nsem, csem, ng0, ng1, po0, po1):
    ngsems = (ng0, ng1)
    posems = (po0, po1)
    wid = _worker_id()
    tbase = wid * TPW
    nbase = wid * NEGPW

    pltpu.async_copy(nodeg_hbm.at[pl.ds(tbase, TPW)], node_v, nsem)

    pltpu.sync_copy(negidx_hbm.at[pl.ds(nbase, NEGPW)], negidx_v)

    @pl.loop(0, NEGPW // CH)
    def _compose(c):
        pltpu.async_copy(walk_hbm.at[negidx_v.at[pl.ds(c * CH, CH)]],
                         nid_v.at[pl.ds(c * CH, CH)], csem)

    pltpu.make_async_copy(
        walk_hbm.at[pl.ds(0, NEGPW)], nid_v, csem).wait()

    def neg_gather(c, b):
        pltpu.async_copy(
            ctx_hbm.at[nid_v.at[pl.ds(c * NEG_CH, NEG_CH)]],
            nrow_v.at[b], ngsems[b])

    for c in range(2):
        neg_gather(c, c)

    pltpu.make_async_copy(
        nodeg_hbm.at[pl.ds(0, TPW)], node_v, nsem).wait()

    @pl.loop(0, NCH_NEG // 2)
    def _neg_group(g):
        for cc in range(2):
            c = g * 2 + cc
            pltpu.make_async_copy(
                ctx_hbm.at[pl.ds(0, NEG_CH)], nrow_v.at[cc], ngsems[cc]).wait()

            @pl.when(g > 0)
            def _():
                pltpu.make_async_copy(
                    pp_v.at[cc], pp_hbm.at[pl.ds(0, NTOK_CH)],
                    posems[cc]).wait()

            @pl.loop(0, NTOK_CH, unroll=2)
            def _tok(tt):
                trow = c * NTOK_CH + tt
                nd = [node_v[trow, pl.ds(q * 16, 16)] for q in range(8)]
                for k in range(NEG):
                    r = tt * NEG + k
                    acc0 = nd[0] * nrow_v[cc, r, pl.ds(0, 16)]
                    acc1 = nd[1] * nrow_v[cc, r, pl.ds(16, 16)]
                    for q in range(2, 8, 2):
                        acc0 += nd[q] * nrow_v[cc, r, pl.ds(q * 16, 16)]
                        acc1 += nd[q + 1] * nrow_v[cc, r, pl.ds(q * 16 + 16, 16)]
                    pp_v[cc, tt, pl.ds(k * 16, 16)] = acc0 + acc1

            pltpu.async_copy(
                pp_v.at[cc],
                pp_hbm.at[pl.ds(tbase + c * NTOK_CH, NTOK_CH)], posems[cc])
            nc = c + 2

            @pl.when(nc < NCH_NEG)
            def _():
                neg_gather(nc, cc)

    for cc in range(2):
        pltpu.make_async_copy(
            pp_v.at[cc], pp_hbm.at[pl.ds(0, NTOK_CH)], posems[cc]).wait()


@functools.cache
def _sc_neg():
    return pl.kernel(
        _sc_neg_body,
        out_type=jax.ShapeDtypeStruct((T, D), jnp.float32),
        mesh=plsc.VectorSubcoreMesh(**_SC_MESH),
        scratch_types=(
            pltpu.VMEM((NEGPW,), jnp.int32),
            pltpu.VMEM((NEGPW,), jnp.int32),
            pltpu.VMEM((TPW, D), jnp.float32),
            pltpu.VMEM((2, NEG_CH, D), jnp.float32),
            pltpu.VMEM((2, NTOK_CH, D), jnp.float32),
        ) + (pltpu.SemaphoreType.DMA,) * 6,
    )


TC_GRID = 32
TB = T // TC_GRID
WPB = TB // WL
NEG_GRID = 8
PPB = T // NEG_GRID


def _nls(score):
    return jnp.log1p(jnp.exp(-jnp.clip(score, -6.0, 6.0)))


def _tc_pos_body(node_ref, ctx_ref, acc_ref):
    i = pl.program_id(0)
    nb = node_ref[...].astype(jnp.bfloat16)
    cb = ctx_ref[...].astype(jnp.bfloat16)
    jj = lax.broadcasted_iota(jnp.int32, (WL, WL), 0)
    ii = lax.broadcasted_iota(jnp.int32, (WL, WL), 1)
    dd = ii - jj
    band = (dd != 0) & (dd >= -WIN) & (dd <= WIN)
    pos_sum = jnp.float32(0.0)
    for w in range(WPB):
        a = nb[w * WL:(w + 1) * WL]
        b = cb[w * WL:(w + 1) * WL]
        g = lax.dot_general(a, b, (((1,), (1,)), ((), ())),
                            preferred_element_type=jnp.float32)
        pos_sum += jnp.sum(jnp.where(band, _nls(g), 0.0))

    @pl.when(i == 0)
    def _init():
        acc_ref[0, 0] = jnp.float32(0.0)

    acc_ref[0, 0] += pos_sum


_tc_pos = pl.pallas_call(
    _tc_pos_body,
    grid=(TC_GRID,),
    in_specs=[
        pl.BlockSpec((TB, D), lambda i: (i, 0)),
        pl.BlockSpec((TB, D), lambda i: (i, 0)),
    ],
    out_specs=pl.BlockSpec(memory_space=pltpu.SMEM),
    out_shape=jax.ShapeDtypeStruct((1, 1), jnp.float32),
)


def _tc_neg_body(pp_ref, acc_ref):
    i = pl.program_id(0)
    seg = (lax.broadcasted_iota(jnp.int32, (D, 8), 0) // 16
           == lax.broadcasted_iota(jnp.int32, (D, 8), 1)).astype(jnp.float32)
    s8 = lax.dot_general(pp_ref[...], seg, (((1,), (0,)), ((), ())),
                         preferred_element_type=jnp.float32)
    kk = lax.broadcasted_iota(jnp.int32, (PPB, 8), 1)
    neg_sum = jnp.sum(jnp.where(kk < NEG, _nls(-s8), 0.0))

    @pl.when(i == 0)
    def _init():
        acc_ref[0, 0] = jnp.float32(0.0)

    acc_ref[0, 0] += neg_sum


_tc_neg = pl.pallas_call(
    _tc_neg_body,
    grid=(NEG_GRID,),
    in_specs=[pl.BlockSpec((PPB, D), lambda i: (i, 0))],
    out_specs=pl.BlockSpec(memory_space=pltpu.SMEM),
    out_shape=jax.ShapeDtypeStruct((1, 1), jnp.float32),
)


def kernel(batch_walk, neg_idx_list_dst, node_embed_weight, context_embed_weight):
    flat_walk = batch_walk.reshape(-1)
    node_g, ctx_g = _sc_tok()(flat_walk, node_embed_weight, context_embed_weight)
    pp = _sc_neg()(flat_walk, neg_idx_list_dst, context_embed_weight, node_g)
    pos_acc = _tc_pos(node_g, ctx_g)
    neg_acc = _tc_neg(pp)
    pos_loss = pos_acc[0, 0] / NPOS
    neg_loss = neg_acc[0, 0] * (NEG * 1.0) / NNEG
    return pos_loss + neg_loss


# speedup vs baseline: 1.1014x; 1.0843x over previous
import functools

import jax
import jax.numpy as jnp
from jax import lax
from jax.experimental import pallas as pl
from jax.experimental.pallas import tpu as pltpu
from jax.experimental.pallas import tpu_sc as plsc

D = 128
B = 512
WL = 40
WIN = 5
NEG = 5
T = B * WL
NNEG = T * NEG
NPOS = B * 2 * sum(WL - d for d in range(1, WIN + 1))

NC = 2
NS = 16
NW = NC * NS
TPW = T // NW
NEGPW = NNEG // NW
CH = 128
NCH_TOK = TPW // CH

NTOK_CH = 16
NEG_CH = NTOK_CH * NEG
NCH_NEG = TPW // NTOK_CH
PPR = NEG_CH // 8
PPG = 4
PP_ROWS = NNEG // 8

_SC_MESH = dict(core_axis_name="c", subcore_axis_name="s",
                num_cores=NC, num_subcores=NS)


def _worker_id():
    return lax.axis_index("s") * NC + lax.axis_index("c")


def _sc_tok_body(walk_hbm, node_hbm, ctx_hbm, nodeg_hbm, ctxg_hbm,
                 walk_v, rows_v, gs0, gs1, gs2, gs3, os0, os1, os2, os3):
    gsems = (gs0, gs1, gs2, gs3)
    osems = (os0, os1, os2, os3)
    wid = _worker_id()
    tbase = wid * TPW

    pltpu.sync_copy(walk_hbm.at[pl.ds(tbase, TPW)], walk_v)

    def gather(j, b):
        idx = walk_v.at[pl.ds((j // 2) * CH, CH)]
        tab = node_hbm if j % 2 == 0 else ctx_hbm
        pltpu.async_copy(tab.at[idx], rows_v.at[b], gsems[b])

    def out(j, b):
        dst = nodeg_hbm if j % 2 == 0 else ctxg_hbm
        pltpu.async_copy(
            rows_v.at[b], dst.at[pl.ds(tbase + (j // 2) * CH, CH)], osems[b])

    for j in range(4):
        gather(j, j)
    for j in range(2 * NCH_TOK):
        b = j % 4
        pltpu.make_async_copy(
            node_hbm.at[pl.ds(0, CH)], rows_v.at[b], gsems[b]).wait()
        out(j, b)
        pltpu.make_async_copy(
            rows_v.at[b], nodeg_hbm.at[pl.ds(0, CH)], osems[b]).wait()
        if j + 4 < 2 * NCH_TOK:
            gather(j + 4, b)


@functools.cache
def _sc_tok():
    return pl.kernel(
        _sc_tok_body,
        out_type=(
            jax.ShapeDtypeStruct((T, D), jnp.float32),
            jax.ShapeDtypeStruct((T, D), jnp.float32),
        ),
        mesh=plsc.VectorSubcoreMesh(**_SC_MESH),
        scratch_types=(
            pltpu.VMEM((TPW,), jnp.int32),
            pltpu.VMEM((4, CH, D), jnp.float32),
        ) + (pltpu.SemaphoreType.DMA,) * 8,
    )


def _sc_neg_body(negidx_hbm, ctxg_hbm, nodeg_hbm, pp_hbm,
                 negidx_v, node_v, nrow_v, pp_v,
                 nsem, ng0, ng1, po0, po1):
    ngsems = (ng0, ng1)
    posems = (po0, po1)
    wid = _worker_id()
    tbase = wid * TPW
    nbase = wid * NEGPW

    pltpu.async_copy(nodeg_hbm.at[pl.ds(tbase, TPW)], node_v, nsem)

    pltpu.sync_copy(negidx_hbm.at[pl.ds(nbase, NEGPW)], negidx_v)

    def neg_gather(c, b):
        pltpu.async_copy(
            ctxg_hbm.at[negidx_v.at[pl.ds(c * NEG_CH, NEG_CH)]],
            nrow_v.at[b], ngsems[b])

    for c in range(2):
        neg_gather(c, c)

    pltpu.make_async_copy(
        nodeg_hbm.at[pl.ds(0, TPW)], node_v, nsem).wait()

    @pl.loop(0, NCH_NEG // 2)
    def _neg_group(g):
        for cc in range(2):
            c = g * 2 + cc
            pltpu.make_async_copy(
                ctxg_hbm.at[pl.ds(0, NEG_CH)], nrow_v.at[cc], ngsems[cc]).wait()

            @pl.when(g > 0)
            def _():
                pltpu.make_async_copy(
                    pp_v.at[cc], pp_hbm.at[pl.ds(0, NTOK_CH)],
                    posems[cc]).wait()

            @pl.loop(0, NTOK_CH, unroll=2)
            def _tok(tt):
                trow = c * NTOK_CH + tt
                nd = [node_v[trow, pl.ds(q * 16, 16)] for q in range(8)]
                for k in range(NEG):
                    r = tt * NEG + k
                    acc0 = nd[0] * nrow_v[cc, r, pl.ds(0, 16)]
                    acc1 = nd[1] * nrow_v[cc, r, pl.ds(16, 16)]
                    for q in range(2, 8, 2):
                        acc0 += nd[q] * nrow_v[cc, r, pl.ds(q * 16, 16)]
                        acc1 += nd[q + 1] * nrow_v[cc, r, pl.ds(q * 16 + 16, 16)]
                    pp_v[cc, tt, pl.ds(k * 16, 16)] = acc0 + acc1

            pltpu.async_copy(
                pp_v.at[cc],
                pp_hbm.at[pl.ds(tbase + c * NTOK_CH, NTOK_CH)], posems[cc])
            nc = c + 2

            @pl.when(nc < NCH_NEG)
            def _():
                neg_gather(nc, cc)

    for cc in range(2):
        pltpu.make_async_copy(
            pp_v.at[cc], pp_hbm.at[pl.ds(0, NTOK_CH)], posems[cc]).wait()


@functools.cache
def _sc_neg():
    return pl.kernel(
        _sc_neg_body,
        out_type=jax.ShapeDtypeStruct((T, D), jnp.float32),
        mesh=plsc.VectorSubcoreMesh(**_SC_MESH),
        scratch_types=(
            pltpu.VMEM((NEGPW,), jnp.int32),
            pltpu.VMEM((TPW, D), jnp.float32),
            pltpu.VMEM((2, NEG_CH, D), jnp.float32),
            pltpu.VMEM((2, NTOK_CH, D), jnp.float32),
        ) + (pltpu.SemaphoreType.DMA,) * 5,
    )


TC_GRID = 32
TB = T // TC_GRID
WPB = TB // WL
NEG_GRID = 8
PPB = T // NEG_GRID


def _nls(score):
    return jnp.log1p(jnp.exp(-jnp.clip(score, -6.0, 6.0)))


def _tc_pos_body(node_ref, ctx_ref, acc_ref):
    i = pl.program_id(0)
    nb = node_ref[...].astype(jnp.bfloat16)
    cb = ctx_ref[...].astype(jnp.bfloat16)
    jj = lax.broadcasted_iota(jnp.int32, (WL, WL), 0)
    ii = lax.broadcasted_iota(jnp.int32, (WL, WL), 1)
    dd = ii - jj
    band = (dd != 0) & (dd >= -WIN) & (dd <= WIN)
    pos_sum = jnp.float32(0.0)
    for w in range(WPB):
        a = nb[w * WL:(w + 1) * WL]
        b = cb[w * WL:(w + 1) * WL]
        g = lax.dot_general(a, b, (((1,), (1,)), ((), ())),
                            preferred_element_type=jnp.float32)
        pos_sum += jnp.sum(jnp.where(band, _nls(g), 0.0))

    @pl.when(i == 0)
    def _init():
        acc_ref[0, 0] = jnp.float32(0.0)

    acc_ref[0, 0] += pos_sum


_tc_pos = pl.pallas_call(
    _tc_pos_body,
    grid=(TC_GRID,),
    in_specs=[
        pl.BlockSpec((TB, D), lambda i: (i, 0)),
        pl.BlockSpec((TB, D), lambda i: (i, 0)),
    ],
    out_specs=pl.BlockSpec(memory_space=pltpu.SMEM),
    out_shape=jax.ShapeDtypeStruct((1, 1), jnp.float32),
)


def _tc_neg_body(pp_ref, acc_ref):
    i = pl.program_id(0)
    seg = (lax.broadcasted_iota(jnp.int32, (D, 8), 0) // 16
           == lax.broadcasted_iota(jnp.int32, (D, 8), 1)).astype(jnp.float32)
    s8 = lax.dot_general(pp_ref[...], seg, (((1,), (0,)), ((), ())),
                         preferred_element_type=jnp.float32)
    kk = lax.broadcasted_iota(jnp.int32, (PPB, 8), 1)
    neg_sum = jnp.sum(jnp.where(kk < NEG, _nls(-s8), 0.0))

    @pl.when(i == 0)
    def _init():
        acc_ref[0, 0] = jnp.float32(0.0)

    acc_ref[0, 0] += neg_sum


_tc_neg = pl.pallas_call(
    _tc_neg_body,
    grid=(NEG_GRID,),
    in_specs=[pl.BlockSpec((PPB, D), lambda i: (i, 0))],
    out_specs=pl.BlockSpec(memory_space=pltpu.SMEM),
    out_shape=jax.ShapeDtypeStruct((1, 1), jnp.float32),
)


def kernel(batch_walk, neg_idx_list_dst, node_embed_weight, context_embed_weight):
    flat_walk = batch_walk.reshape(-1)
    node_g, ctx_g = _sc_tok()(flat_walk, node_embed_weight, context_embed_weight)
    pp = _sc_neg()(neg_idx_list_dst, ctx_g, node_g)
    pos_acc = _tc_pos(node_g, ctx_g)
    neg_acc = _tc_neg(pp)
    pos_loss = pos_acc[0, 0] / NPOS
    neg_loss = neg_acc[0, 0] * (NEG * 1.0) / NNEG
    return pos_loss + neg_loss
